# Initial kernel scaffold; baseline (speedup 1.0000x reference)
import jax
import jax.numpy as jnp
from jax.experimental import pallas as pl


def kernel(x, Wte, bte, l2_emb, cl_emb, Wg, bg, W1, b1, W2, b2, ln_l2_g, ln_l2_b, ln_cl_g, ln_cl_b, Wt2i, bt2i, Wcl, bcl):
    raise NotImplementedError("write your pallas kernel here")


# speedup vs baseline: 7.5834x; 7.5834x over previous
import functools

import jax
import jax.numpy as jnp
from jax.experimental import pallas as pl

_SQRT_HALF = 0.7071067811865476


def _gelu(x):
    return 0.5 * x * (1.0 + jax.lax.erf(x * _SQRT_HALF))


def _top2_weights(p, E):
    T = p.shape[0]
    iota = jax.lax.broadcasted_iota(jnp.int32, (T, E), 1)
    m1 = jnp.max(p, axis=-1, keepdims=True)
    i1 = jnp.min(jnp.where(p == m1, iota, E), axis=-1, keepdims=True)
    sel1 = iota == i1
    p2 = jnp.where(sel1, -1.0, p)
    m2 = jnp.max(p2, axis=-1, keepdims=True)
    i2 = jnp.min(jnp.where(p2 == m2, iota, E), axis=-1, keepdims=True)
    sel2 = iota == i2
    return jnp.where(sel1 | sel2, p, 0.0)


def _dot(a, b):
    return jnp.dot(a, b, preferred_element_type=jnp.float32)


def _moe_kernel(x_ref, Wte_ref, bte_ref, l2e_ref, cle_ref, Wg_ref, bg_ref,
                W1c_ref, b1f_ref, W2s_ref, b2_ref, lnl2g_ref, lnl2b_ref,
                lncg_ref, lncb_ref, Wt2i_ref, bt2i_ref, Wcl_ref, bcl_ref,
                l2r_ref, clr_ref, *, E, H):
    xh = _gelu(_dot(x_ref[...], Wte_ref[...]) + bte_ref[...])
    T = xh.shape[0]
    rep = (jax.lax.broadcasted_iota(jnp.int32, (E, E * H), 1) // H
           == jax.lax.broadcasted_iota(jnp.int32, (E, E * H), 0)).astype(jnp.float32)

    def branch(emb_ref, g_ref, b_ref):
        inp = xh + emb_ref[...]
        logits = _dot(inp, Wg_ref[...]) + bg_ref[...]
        m = jnp.max(logits, axis=-1, keepdims=True)
        ex = jnp.exp(logits - m)
        p = ex / jnp.sum(ex, axis=-1, keepdims=True)
        w = _top2_weights(p, E)
        h = _gelu(_dot(inp, W1c_ref[...]) + b1f_ref[...])
        wrep = _dot(w, rep)
        moe = _dot(h * wrep, W2s_ref[...]) + _dot(w, b2_ref[...])
        mu = jnp.mean(moe, axis=-1, keepdims=True)
        var = jnp.mean((moe - mu) ** 2, axis=-1, keepdims=True)
        ln = g_ref[...] * (moe - mu) * jax.lax.rsqrt(var + 1e-5) + b_ref[...]
        return _gelu(ln) + inp

    l2o = branch(l2e_ref, lnl2g_ref, lnl2b_ref)
    clo = branch(cle_ref, lncg_ref, lncb_ref)
    l2r_ref[...] = _dot(l2o, Wt2i_ref[...]) + bt2i_ref[...]
    clr_ref[...] = _dot(clo, Wcl_ref[...]) + bcl_ref[...]


def kernel(x, Wte, bte, l2_emb, cl_emb, Wg, bg, W1, b1, W2, b2,
           ln_l2_g, ln_l2_b, ln_cl_g, ln_cl_b, Wt2i, bt2i, Wcl, bcl):
    B, S, TD = x.shape
    H = Wte.shape[1]
    E = Wg.shape[1]
    N = B * S
    T = min(512, N)
    xf = x.reshape(N, TD)
    W1c = W1.transpose(1, 0, 2).reshape(H, E * H)
    b1f = b1.reshape(1, E * H)
    W2s = W2.reshape(E * H, H)

    row = lambda v: v.reshape(1, -1)
    full = lambda shape: pl.BlockSpec(shape, lambda i: (0, 0))

    grid = (N // T,)
    out = pl.pallas_call(
        functools.partial(_moe_kernel, E=E, H=H),
        grid=grid,
        in_specs=[
            pl.BlockSpec((T, TD), lambda i: (i, 0)),
            full((TD, H)), full((1, H)), full((1, H)), full((1, H)),
            full((H, E)), full((1, E)),
            full((H, E * H)), full((1, E * H)),
            full((E * H, H)), full((E, H)),
            full((1, H)), full((1, H)), full((1, H)), full((1, H)),
            full((H, TD)), full((1, TD)),
            full((H, H)), full((1, H)),
        ],
        out_specs=[
            pl.BlockSpec((T, TD), lambda i: (i, 0)),
            pl.BlockSpec((T, H), lambda i: (i, 0)),
        ],
        out_shape=[
            jax.ShapeDtypeStruct((N, TD), jnp.float32),
            jax.ShapeDtypeStruct((N, H), jnp.float32),
        ],
    )(xf, Wte, row(bte), row(l2_emb), row(cl_emb), Wg, row(bg),
      W1c, b1f, W2s, b2, row(ln_l2_g), row(ln_l2_b), row(ln_cl_g),
      row(ln_cl_b), Wt2i, row(bt2i), Wcl, row(bcl))
    l2r, clr = out
    return (l2r.reshape(B, S, TD), clr.reshape(B, S, H))


# speedup vs baseline: 7.6043x; 1.0028x over previous
import functools

import jax
import jax.numpy as jnp
from jax.experimental import pallas as pl

_SQRT_HALF = 0.7071067811865476


def _gelu(x):
    return 0.5 * x * (1.0 + jax.lax.erf(x * _SQRT_HALF))


def _top2_weights(p, E):
    T = p.shape[0]
    iota = jax.lax.broadcasted_iota(jnp.int32, (T, E), 1)
    m1 = jnp.max(p, axis=-1, keepdims=True)
    i1 = jnp.min(jnp.where(p == m1, iota, E), axis=-1, keepdims=True)
    sel1 = iota == i1
    p2 = jnp.where(sel1, -1.0, p)
    m2 = jnp.max(p2, axis=-1, keepdims=True)
    i2 = jnp.min(jnp.where(p2 == m2, iota, E), axis=-1, keepdims=True)
    sel2 = iota == i2
    return jnp.where(sel1 | sel2, p, 0.0)


def _dot(a, b):
    return jnp.dot(a, b, preferred_element_type=jnp.float32)


def _bdot(a, b):
    return jnp.dot(a.astype(jnp.bfloat16), b, preferred_element_type=jnp.float32)


def _moe_kernel(x_ref, Wte_ref, bte_ref, l2e_ref, cle_ref, Wg_ref, bg_ref,
                W1c_ref, b1f_ref, W2s_ref, b2_ref, lnl2g_ref, lnl2b_ref,
                lncg_ref, lncb_ref, Wt2i_ref, bt2i_ref, Wcl_ref, bcl_ref,
                l2r_ref, clr_ref, *, E, H):
    xh = _gelu(_dot(x_ref[...], Wte_ref[...]) + bte_ref[...])
    T = xh.shape[0]
    rep = (jax.lax.broadcasted_iota(jnp.int32, (E, E * H), 1) // H
           == jax.lax.broadcasted_iota(jnp.int32, (E, E * H), 0)).astype(jnp.float32)

    def branch(emb_ref, g_ref, b_ref):
        inp = xh + emb_ref[...]
        logits = _dot(inp, Wg_ref[...]) + bg_ref[...]
        m = jnp.max(logits, axis=-1, keepdims=True)
        ex = jnp.exp(logits - m)
        p = ex / jnp.sum(ex, axis=-1, keepdims=True)
        w = _top2_weights(p, E)
        h = _gelu(_bdot(inp, W1c_ref[...]) + b1f_ref[...])
        wrep = _dot(w, rep)
        moe = _bdot(h * wrep, W2s_ref[...]) + _dot(w, b2_ref[...])
        mu = jnp.mean(moe, axis=-1, keepdims=True)
        var = jnp.mean((moe - mu) ** 2, axis=-1, keepdims=True)
        ln = g_ref[...] * (moe - mu) * jax.lax.rsqrt(var + 1e-5) + b_ref[...]
        return _gelu(ln) + inp

    l2o = branch(l2e_ref, lnl2g_ref, lnl2b_ref)
    clo = branch(cle_ref, lncg_ref, lncb_ref)
    l2r_ref[...] = _bdot(l2o, Wt2i_ref[...]) + bt2i_ref[...]
    clr_ref[...] = _bdot(clo, Wcl_ref[...]) + bcl_ref[...]


def kernel(x, Wte, bte, l2_emb, cl_emb, Wg, bg, W1, b1, W2, b2,
           ln_l2_g, ln_l2_b, ln_cl_g, ln_cl_b, Wt2i, bt2i, Wcl, bcl):
    B, S, TD = x.shape
    H = Wte.shape[1]
    E = Wg.shape[1]
    N = B * S
    T = min(512, N)
    xf = x.reshape(N, TD)
    W1c = W1.transpose(1, 0, 2).reshape(H, E * H).astype(jnp.bfloat16)
    b1f = b1.reshape(1, E * H)
    W2s = W2.reshape(E * H, H).astype(jnp.bfloat16)
    Wt2i = Wt2i.astype(jnp.bfloat16)
    Wcl = Wcl.astype(jnp.bfloat16)

    row = lambda v: v.reshape(1, -1)
    full = lambda shape: pl.BlockSpec(shape, lambda i: (0, 0))

    grid = (N // T,)
    out = pl.pallas_call(
        functools.partial(_moe_kernel, E=E, H=H),
        grid=grid,
        in_specs=[
            pl.BlockSpec((T, TD), lambda i: (i, 0)),
            full((TD, H)), full((1, H)), full((1, H)), full((1, H)),
            full((H, E)), full((1, E)),
            full((H, E * H)), full((1, E * H)),
            full((E * H, H)), full((E, H)),
            full((1, H)), full((1, H)), full((1, H)), full((1, H)),
            full((H, TD)), full((1, TD)),
            full((H, H)), full((1, H)),
        ],
        out_specs=[
            pl.BlockSpec((T, TD), lambda i: (i, 0)),
            pl.BlockSpec((T, H), lambda i: (i, 0)),
        ],
        out_shape=[
            jax.ShapeDtypeStruct((N, TD), jnp.float32),
            jax.ShapeDtypeStruct((N, H), jnp.float32),
        ],
    )(xf, Wte, row(bte), row(l2_emb), row(cl_emb), Wg, row(bg),
      W1c, b1f, W2s, b2, row(ln_l2_g), row(ln_l2_b), row(ln_cl_g),
      row(ln_cl_b), Wt2i, row(bt2i), Wcl, row(bcl))
    l2r, clr = out
    return (l2r.reshape(B, S, TD), clr.reshape(B, S, H))


# speedup vs baseline: 8.7368x; 1.1489x over previous
import functools

import jax
import jax.numpy as jnp
from jax.experimental import pallas as pl

_SQRT_HALF = 0.7071067811865476


def _gelu(x):
    return 0.5 * x * (1.0 + jax.lax.erf(x * _SQRT_HALF))


def _dot(a, b):
    return jnp.dot(a, b, preferred_element_type=jnp.float32)


def _bdot(a, b):
    return jnp.dot(a.astype(jnp.bfloat16), b, preferred_element_type=jnp.float32)


def _top2_weights_t(lt, E):
    T = lt.shape[1]
    m = jnp.max(lt, axis=0, keepdims=True)
    ex = jnp.exp(lt - m)
    z = jnp.sum(ex, axis=0, keepdims=True)
    iota = jax.lax.broadcasted_iota(jnp.int32, (E, T), 0)
    i1 = jnp.min(jnp.where(lt == m, iota, E), axis=0, keepdims=True)
    sel1 = iota == i1
    lt2 = jnp.where(sel1, -jnp.inf, lt)
    m2 = jnp.max(lt2, axis=0, keepdims=True)
    i2 = jnp.min(jnp.where(lt2 == m2, iota, E), axis=0, keepdims=True)
    sel2 = iota == i2
    return jnp.where(sel1 | sel2, ex / z, 0.0)


def _moe_kernel(x_ref, Wte_ref, bte_ref, l2e_ref, cle_ref, Wg_ref, bg_ref,
                W1c_ref, b1f_ref, W2s_ref, b2_ref, rep_ref, lnl2g_ref,
                lnl2b_ref, lncg_ref, lncb_ref, Wt2i_ref, bt2i_ref, Wcl_ref,
                bcl_ref, l2r_ref, clr_ref, *, E, H):
    xh = _gelu(_dot(x_ref[...], Wte_ref[...]) + bte_ref[...])

    def branch(emb_ref, g_ref, b_ref):
        inp = xh + emb_ref[...]
        logits = _dot(inp, Wg_ref[...]) + bg_ref[...]
        wt = _top2_weights_t(logits.T, E)
        w = wt.T
        zpre = _bdot(inp, W1c_ref[...]) + b1f_ref[...]
        h = _gelu(zpre.astype(jnp.bfloat16))
        wrep = _bdot(w, rep_ref[...]).astype(jnp.bfloat16)
        moe = _dot(h * wrep, W2s_ref[...]) + _dot(w, b2_ref[...])
        mu = jnp.mean(moe, axis=-1, keepdims=True)
        var = jnp.mean((moe - mu) ** 2, axis=-1, keepdims=True)
        ln = g_ref[...] * (moe - mu) * jax.lax.rsqrt(var + 1e-5) + b_ref[...]
        return _gelu(ln) + inp

    l2o = branch(l2e_ref, lnl2g_ref, lnl2b_ref)
    clo = branch(cle_ref, lncg_ref, lncb_ref)
    l2r_ref[...] = _bdot(l2o, Wt2i_ref[...]) + bt2i_ref[...]
    clr_ref[...] = _bdot(clo, Wcl_ref[...]) + bcl_ref[...]


def kernel(x, Wte, bte, l2_emb, cl_emb, Wg, bg, W1, b1, W2, b2,
           ln_l2_g, ln_l2_b, ln_cl_g, ln_cl_b, Wt2i, bt2i, Wcl, bcl):
    B, S, TD = x.shape
    H = Wte.shape[1]
    E = Wg.shape[1]
    N = B * S
    T = min(512, N)
    xf = x.reshape(N, TD)
    W1c = W1.transpose(1, 0, 2).reshape(H, E * H).astype(jnp.bfloat16)
    b1f = b1.reshape(1, E * H)
    W2s = W2.reshape(E * H, H).astype(jnp.bfloat16)
    Wt2i = Wt2i.astype(jnp.bfloat16)
    Wcl = Wcl.astype(jnp.bfloat16)
    rep = jnp.repeat(jnp.eye(E, dtype=jnp.bfloat16), H, axis=1)

    row = lambda v: v.reshape(1, -1)
    full = lambda shape: pl.BlockSpec(shape, lambda i: (0, 0))

    grid = (N // T,)
    out = pl.pallas_call(
        functools.partial(_moe_kernel, E=E, H=H),
        grid=grid,
        in_specs=[
            pl.BlockSpec((T, TD), lambda i: (i, 0)),
            full((TD, H)), full((1, H)), full((1, H)), full((1, H)),
            full((H, E)), full((1, E)),
            full((H, E * H)), full((1, E * H)),
            full((E * H, H)), full((E, H)), full((E, E * H)),
            full((1, H)), full((1, H)), full((1, H)), full((1, H)),
            full((H, TD)), full((1, TD)),
            full((H, H)), full((1, H)),
        ],
        out_specs=[
            pl.BlockSpec((T, TD), lambda i: (i, 0)),
            pl.BlockSpec((T, H), lambda i: (i, 0)),
        ],
        out_shape=[
            jax.ShapeDtypeStruct((N, TD), jnp.float32),
            jax.ShapeDtypeStruct((N, H), jnp.float32),
        ],
    )(xf, Wte, row(bte), row(l2_emb), row(cl_emb), Wg, row(bg),
      W1c, b1f, W2s, b2, rep, row(ln_l2_g), row(ln_l2_b), row(ln_cl_g),
      row(ln_cl_b), Wt2i, row(bt2i), Wcl, row(bcl))
    l2r, clr = out
    return (l2r.reshape(B, S, TD), clr.reshape(B, S, H))


# speedup vs baseline: 9.2445x; 1.0581x over previous
import functools

import jax
import jax.numpy as jnp
from jax.experimental import pallas as pl

_SQRT_HALF = 0.7071067811865476


def _gelu(x):
    return 0.5 * x * (1.0 + jax.lax.erf(x * _SQRT_HALF))


def _dot(a, b):
    return jnp.dot(a, b, preferred_element_type=jnp.float32)


def _bdot(a, b):
    return jnp.dot(a.astype(jnp.bfloat16), b, preferred_element_type=jnp.float32)


def _top2_weights_t(lt, E):
    T = lt.shape[1]
    m = jnp.max(lt, axis=0, keepdims=True)
    ex = jnp.exp(lt - m)
    z = jnp.sum(ex, axis=0, keepdims=True)
    iota = jax.lax.broadcasted_iota(jnp.int32, (E, T), 0)
    i1 = jnp.min(jnp.where(lt == m, iota, E), axis=0, keepdims=True)
    sel1 = iota == i1
    lt2 = jnp.where(sel1, -jnp.inf, lt)
    m2 = jnp.max(lt2, axis=0, keepdims=True)
    i2 = jnp.min(jnp.where(lt2 == m2, iota, E), axis=0, keepdims=True)
    sel2 = iota == i2
    return jnp.where(sel1 | sel2, ex / z, 0.0)


def _moe_kernel(x_ref, Wte_ref, bte_ref, l2e_ref, cle_ref, Wg_ref, bg_ref,
                W1c_ref, b1f_ref, W2s_ref, b2_ref, rep_ref, lnl2g_ref,
                lnl2b_ref, lncg_ref, lncb_ref, Wt2i_ref, bt2i_ref, Wcl_ref,
                bcl_ref, l2r_ref, clr_ref, *, E, H):
    xh = _gelu(_dot(x_ref[...], Wte_ref[...]) + bte_ref[...])

    def branch(emb_ref, g_ref, b_ref):
        inp = xh + emb_ref[...]
        logits = _dot(inp, Wg_ref[...]) + bg_ref[...]
        wt = _top2_weights_t(logits.T, E)
        w = wt.T
        zpre = _bdot(inp, W1c_ref[...]) + b1f_ref[...]
        h = _gelu(zpre.astype(jnp.bfloat16))
        wrep = _bdot(w, rep_ref[...]).astype(jnp.bfloat16)
        moe = _dot(h * wrep, W2s_ref[...]) + _dot(w, b2_ref[...])
        mu = jnp.mean(moe, axis=-1, keepdims=True)
        var = jnp.mean((moe - mu) ** 2, axis=-1, keepdims=True)
        ln = g_ref[...] * (moe - mu) * jax.lax.rsqrt(var + 1e-5) + b_ref[...]
        return _gelu(ln) + inp

    l2o = branch(l2e_ref, lnl2g_ref, lnl2b_ref)
    clo = branch(cle_ref, lncg_ref, lncb_ref)
    l2r_ref[...] = _bdot(l2o, Wt2i_ref[...]) + bt2i_ref[...]
    clr_ref[...] = _bdot(clo, Wcl_ref[...]) + bcl_ref[...]


def kernel(x, Wte, bte, l2_emb, cl_emb, Wg, bg, W1, b1, W2, b2,
           ln_l2_g, ln_l2_b, ln_cl_g, ln_cl_b, Wt2i, bt2i, Wcl, bcl):
    B, S, TD = x.shape
    H = Wte.shape[1]
    E = Wg.shape[1]
    N = B * S
    T = min(1024, N)
    xf = x.reshape(N, TD)
    W1c = W1.transpose(1, 0, 2).reshape(H, E * H).astype(jnp.bfloat16)
    b1f = b1.reshape(1, E * H)
    W2s = W2.reshape(E * H, H).astype(jnp.bfloat16)
    Wt2i = Wt2i.astype(jnp.bfloat16)
    Wcl = Wcl.astype(jnp.bfloat16)
    rep = jnp.repeat(jnp.eye(E, dtype=jnp.bfloat16), H, axis=1)

    row = lambda v: v.reshape(1, -1)
    full = lambda shape: pl.BlockSpec(shape, lambda i: (0, 0))

    grid = (N // T,)
    out = pl.pallas_call(
        functools.partial(_moe_kernel, E=E, H=H),
        grid=grid,
        in_specs=[
            pl.BlockSpec((T, TD), lambda i: (i, 0)),
            full((TD, H)), full((1, H)), full((1, H)), full((1, H)),
            full((H, E)), full((1, E)),
            full((H, E * H)), full((1, E * H)),
            full((E * H, H)), full((E, H)), full((E, E * H)),
            full((1, H)), full((1, H)), full((1, H)), full((1, H)),
            full((H, TD)), full((1, TD)),
            full((H, H)), full((1, H)),
        ],
        out_specs=[
            pl.BlockSpec((T, TD), lambda i: (i, 0)),
            pl.BlockSpec((T, H), lambda i: (i, 0)),
        ],
        out_shape=[
            jax.ShapeDtypeStruct((N, TD), jnp.float32),
            jax.ShapeDtypeStruct((N, H), jnp.float32),
        ],
    )(xf, Wte, row(bte), row(l2_emb), row(cl_emb), Wg, row(bg),
      W1c, b1f, W2s, b2, rep, row(ln_l2_g), row(ln_l2_b), row(ln_cl_g),
      row(ln_cl_b), Wt2i, row(bt2i), Wcl, row(bcl))
    l2r, clr = out
    return (l2r.reshape(B, S, TD), clr.reshape(B, S, H))
